# baseline (speedup 1.0000x reference)
import jax
import jax.numpy as jnp
from jax.experimental import pallas as pl
from jax.experimental.pallas import tpu as pltpu

BATCH = 4096
SEQ = 200
DIM = 64
VOCAB = 1000
VPAD = 1024
TBM = 16
R = TBM * SEQ


def _tc_body(x_ref, table_ref, pos_ref, out_ref):
    classes = jax.lax.broadcasted_iota(jnp.int32, (R, VPAD), 1)
    oh = (classes == x_ref[...]).astype(jnp.bfloat16)
    acc = jnp.dot(oh, table_ref[...], preferred_element_type=jnp.float32)
    out_ref[...] = acc + pos_ref[...]


def kernel(X, nucleo_table, pos_table):
    xf = X.reshape(BATCH * SEQ, 1)
    table_bf = jnp.pad(nucleo_table, ((0, VPAD - VOCAB), (0, 0))).astype(
        jnp.bfloat16)
    pos_rep = jnp.tile(pos_table, (TBM, 1))
    grid = (BATCH // TBM,)
    out = pl.pallas_call(
        _tc_body,
        grid=grid,
        in_specs=[
            pl.BlockSpec((R, 1), lambda i: (i, 0)),
            pl.BlockSpec((VPAD, DIM), lambda i: (0, 0)),
            pl.BlockSpec((R, DIM), lambda i: (0, 0)),
        ],
        out_specs=pl.BlockSpec((R, DIM), lambda i: (i, 0)),
        out_shape=jax.ShapeDtypeStruct((BATCH * SEQ, DIM), jnp.float32),
    )(xf, table_bf, pos_rep)
    return out.reshape(BATCH, SEQ, DIM)


# speedup vs baseline: 1.0728x; 1.0728x over previous
import jax
import jax.numpy as jnp
from jax import lax
from jax.experimental import pallas as pl
from jax.experimental.pallas import tpu as pltpu
from jax.experimental.pallas import tpu_sc as plsc

BATCH = 4096
SEQ = 200
DIM = 64
VOCAB = 1000

B_SC = 2560
NC = 2
NS = 16
PG = 4
BG = 8
PP = SEQ // PG
BB = B_SC // BG
TB = 16
NT = BB // TB
NSL = DIM // 16

B_TC = BATCH - B_SC
VPAD = 1024
TBM = 16
R = TBM * SEQ
TC_OFF = B_SC // TBM


def _sc_body(x_hbm, nucleo_hbm, pos_hbm, out_hbm,
             idx0, idx1, buf0, buf1, pos_v, table_sh,
             isem0, isem1, gsem0, gsem1, ssem0, ssem1):
    idxv = (idx0, idx1)
    buf = (buf0, buf1)
    isem = (isem0, isem1)
    gsem = (gsem0, gsem1)
    ssem = (ssem0, ssem1)
    wid = lax.axis_index("s") * NC + lax.axis_index("c")
    wp = wid % PG
    wb = wid // PG
    p0 = wp * PP
    b0 = wb * BB

    @pl.when(lax.axis_index("s") == 0)
    def _stage_table():
        pltpu.sync_copy(nucleo_hbm, table_sh)

    plsc.subcore_barrier()
    pltpu.sync_copy(pos_hbm.at[pl.ds(p0, PP)], pos_v)

    def stage_idx(i, s):
        pltpu.async_copy(
            x_hbm.at[wp, pl.ds(b0 + i * TB, TB)], idxv[s], isem[s])

    def fire_gathers(s):
        for n in range(TB):
            pltpu.async_copy(
                table_sh.at[idxv[s].at[n]], buf[s].at[n], gsem[s])

    def out_slice(i):
        return out_hbm.at[pl.ds(b0 + i * TB, TB), pl.ds(p0, PP)]

    stage_idx(0, 0)
    pltpu.make_async_copy(
        x_hbm.at[wp, pl.ds(b0, TB)], idxv[0], isem[0]).wait()
    fire_gathers(0)

    def pair(g, carry):
        for b in range(2):
            i = g * 2 + b
            s, t = b, 1 - b

            @pl.when(i + 1 < NT)
            def _stage():
                stage_idx(i + 1, t)

                @pl.when(i >= 1)
                def _drain_store():
                    pltpu.make_async_copy(
                        buf[t], out_slice(i - 1), ssem[t]).wait()

                pltpu.make_async_copy(
                    x_hbm.at[wp, pl.ds(b0 + (i + 1) * TB, TB)],
                    idxv[t], isem[t]).wait()
                fire_gathers(t)

            for n in range(TB):
                pltpu.make_async_copy(
                    table_sh.at[idxv[s].at[n]], buf[s].at[n],
                    gsem[s]).wait()

            def padd(p, carry2):
                posr = [pos_v[p, pl.ds(j * 16, 16)] for j in range(NSL)]
                for n in range(TB):
                    for j in range(NSL):
                        sl = pl.ds(j * 16, 16)
                        buf[s][n, p, sl] = buf[s][n, p, sl] + posr[j]
                return carry2

            lax.fori_loop(0, PP, padd, 0)
            pltpu.async_copy(buf[s], out_slice(i), ssem[s])
        return carry

    lax.fori_loop(0, NT // 2, pair, 0)

    pltpu.make_async_copy(buf[0], out_slice(NT - 2), ssem[0]).wait()
    pltpu.make_async_copy(buf[1], out_slice(NT - 1), ssem[1]).wait()


def _sc_part(X_sc, nucleo_table, pos_table):
    xb = X_sc.reshape(B_SC, PG, PP).transpose(1, 0, 2)
    mesh = plsc.VectorSubcoreMesh(core_axis_name="c", subcore_axis_name="s")
    k = pl.kernel(
        _sc_body,
        mesh=mesh,
        compiler_params=pltpu.CompilerParams(use_tc_tiling_on_sc=False),
        out_type=jax.ShapeDtypeStruct((B_SC, SEQ, DIM), jnp.float32),
        scratch_types=[
            pltpu.VMEM((TB, PP), jnp.int32),
            pltpu.VMEM((TB, PP), jnp.int32),
            pltpu.VMEM((TB, PP, DIM), jnp.float32),
            pltpu.VMEM((TB, PP, DIM), jnp.float32),
            pltpu.VMEM((PP, DIM), jnp.float32),
            pltpu.VMEM_SHARED((VOCAB, DIM), jnp.float32),
            pltpu.SemaphoreType.DMA,
            pltpu.SemaphoreType.DMA,
            pltpu.SemaphoreType.DMA,
            pltpu.SemaphoreType.DMA,
            pltpu.SemaphoreType.DMA,
            pltpu.SemaphoreType.DMA,
        ],
    )
    return k(xb, nucleo_table, pos_table)


def _tc_body(x_ref, table_ref, pos_ref, out_ref):
    classes = jax.lax.broadcasted_iota(jnp.int32, (R, VPAD), 1)
    oh = (classes == x_ref[...]).astype(jnp.bfloat16)
    acc = jnp.dot(oh, table_ref[...], preferred_element_type=jnp.float32)
    out_ref[...] = acc + pos_ref[...]


def _tc_part(X_tc, nucleo_table, pos_table):
    xf = X_tc.reshape(B_TC * SEQ, 1)
    table_bf = jnp.pad(nucleo_table, ((0, VPAD - VOCAB), (0, 0))).astype(
        jnp.bfloat16)
    pos_rep = jnp.tile(pos_table, (TBM, 1))
    out = pl.pallas_call(
        _tc_body,
        grid=(B_TC // TBM,),
        in_specs=[
            pl.BlockSpec((R, 1), lambda i: (i, 0)),
            pl.BlockSpec((VPAD, DIM), lambda i: (0, 0)),
            pl.BlockSpec((R, DIM), lambda i: (0, 0)),
        ],
        out_specs=pl.BlockSpec((R, DIM), lambda i: (TC_OFF + i, 0)),
        out_shape=jax.ShapeDtypeStruct((BATCH * SEQ, DIM), jnp.float32),
    )(xf, table_bf, pos_rep)
    return out.reshape(BATCH, SEQ, DIM)


def kernel(X, nucleo_table, pos_table):
    sc_out = _sc_part(X[:B_SC], nucleo_table, pos_table)
    full = _tc_part(X[B_SC:], nucleo_table, pos_table)
    return lax.dynamic_update_slice(full, sc_out, (0, 0, 0))
